# baseline (device time: 106030 ns/iter reference)
import jax
import jax.numpy as jnp
from jax import lax
from jax.experimental import pallas as pl
from jax.experimental.pallas import tpu as pltpu

N_DEV = 8
B, SQ, DM = 2, 256, 512
HQ, DH = 32, 64
H_LOC = HQ // N_DEV
DLOC = H_LOC * DH
ROWS = B * SQ
BLK = 64


def kernel(x, Wq, K_ext, V_ext, Wo):
    x2 = x.reshape(ROWS, DM)
    k2 = K_ext.reshape(ROWS, DLOC)
    v2 = V_ext.reshape(ROWS, DLOC)

    def body(x_ref, wq_ref, k_ref, v_ref, wo_ref, out_ref,
             comm_ref, send_sems, recv_sems):
        my = lax.axis_index("i")
        left = lax.rem(my + N_DEV - 1, N_DEV)
        right = lax.rem(my + 1, N_DEV)

        wq = wq_ref[:, pl.ds(my * DLOC, DLOC)]
        q = jnp.dot(x_ref[...], wq, preferred_element_type=jnp.float32)

        qb = lax.broadcasted_iota(jnp.int32, (SQ, SQ), 0) // BLK
        kb = lax.broadcasted_iota(jnp.int32, (SQ, SQ), 1) // BLK
        mask = (qb == kb) | ((kb % 4) == (qb % 4))

        kv = k_ref[...]
        vv = v_ref[...]
        ctx_rows = []
        for b in range(B):
            r0 = b * SQ
            heads = []
            for h in range(H_LOC):
                c0 = h * DH
                q_bh = q[r0:r0 + SQ, c0:c0 + DH]
                k_bh = kv[r0:r0 + SQ, c0:c0 + DH]
                v_bh = vv[r0:r0 + SQ, c0:c0 + DH]
                s = lax.dot_general(
                    q_bh, k_bh, (((1,), (1,)), ((), ())),
                    preferred_element_type=jnp.float32,
                ) * 0.125
                s = jnp.where(mask, s, -1e9)
                m = jnp.max(s, axis=-1, keepdims=True)
                w = jnp.exp(s - m)
                w = w / jnp.sum(w, axis=-1, keepdims=True)
                heads.append(jnp.dot(w, v_bh, preferred_element_type=jnp.float32))
            ctx_rows.append(jnp.concatenate(heads, axis=1))
        ctx = jnp.concatenate(ctx_rows, axis=0)

        wo = wo_ref[pl.ds(my * DLOC, DLOC), :]
        partial = jnp.dot(ctx, wo, preferred_element_type=jnp.float32)

        comm_ref[N_DEV - 1, :, :] = partial
        acc = partial

        barrier_sem = pltpu.get_barrier_semaphore()
        for nbr in (left, right):
            pl.semaphore_signal(
                barrier_sem, inc=1,
                device_id=(nbr,), device_id_type=pl.DeviceIdType.MESH,
            )
        pl.semaphore_wait(barrier_sem, 2)

        for h in range(N_DEV - 1):
            send_slot = N_DEV - 1 if h == 0 else h - 1
            rdma = pltpu.make_async_remote_copy(
                src_ref=comm_ref.at[send_slot],
                dst_ref=comm_ref.at[h],
                send_sem=send_sems.at[h],
                recv_sem=recv_sems.at[h],
                device_id=(right,),
                device_id_type=pl.DeviceIdType.MESH,
            )
            rdma.start()
            rdma.wait()
            acc = acc + comm_ref[h, :, :]

        out_ref[...] = acc

    out2 = pl.pallas_call(
        body,
        out_shape=jax.ShapeDtypeStruct((ROWS, DM), jnp.float32),
        in_specs=[pl.BlockSpec(memory_space=pltpu.VMEM)] * 5,
        out_specs=pl.BlockSpec(memory_space=pltpu.VMEM),
        scratch_shapes=[
            pltpu.VMEM((N_DEV, ROWS, DM), jnp.float32),
            pltpu.SemaphoreType.DMA((N_DEV - 1,)),
            pltpu.SemaphoreType.DMA((N_DEV - 1,)),
        ],
        compiler_params=pltpu.CompilerParams(collective_id=0),
    )(x2, Wq, k2, v2, Wo)
    return out2.reshape(B, SQ, DM)


# device time: 31599 ns/iter; 3.3555x vs baseline; 3.3555x over previous
import jax
import jax.numpy as jnp
from jax import lax
from jax.experimental import pallas as pl
from jax.experimental.pallas import tpu as pltpu

N_DEV = 8
B, SQ, DM = 2, 256, 512
HQ, DH = 32, 64
H_LOC = HQ // N_DEV
DLOC = H_LOC * DH
ROWS = B * SQ
CH = ROWS // N_DEV
BLK = 64


def kernel(x, Wq, K_ext, V_ext, Wo):
    x2 = x.reshape(ROWS, DM)
    k2 = K_ext.reshape(ROWS, DLOC)
    v2 = V_ext.reshape(ROWS, DLOC)

    def body(x_ref, wq_ref, k_ref, v_ref, wo_ref, out_ref,
             part_ref, red_ref, p1_buf,
             p1_send, p1_recv, p2_send, p2_recv):
        my = lax.axis_index("i")

        wq = wq_ref[:, pl.ds(my * DLOC, DLOC)]
        q = jnp.dot(x_ref[...], wq, preferred_element_type=jnp.float32)

        qb = lax.broadcasted_iota(jnp.int32, (SQ, SQ), 0) // BLK
        kb = lax.broadcasted_iota(jnp.int32, (SQ, SQ), 1) // BLK
        mask = (qb == kb) | ((kb % 4) == (qb % 4))

        kv = k_ref[...]
        vv = v_ref[...]
        ctx_rows = []
        for b in range(B):
            r0 = b * SQ
            heads = []
            for h in range(H_LOC):
                c0 = h * DH
                q_bh = q[r0:r0 + SQ, c0:c0 + DH]
                k_bh = kv[r0:r0 + SQ, c0:c0 + DH]
                v_bh = vv[r0:r0 + SQ, c0:c0 + DH]
                s = lax.dot_general(
                    q_bh, k_bh, (((1,), (1,)), ((), ())),
                    preferred_element_type=jnp.float32,
                ) * 0.125
                s = jnp.where(mask, s, -1e9)
                m = jnp.max(s, axis=-1, keepdims=True)
                w = jnp.exp(s - m)
                w = w / jnp.sum(w, axis=-1, keepdims=True)
                heads.append(jnp.dot(w, v_bh, preferred_element_type=jnp.float32))
            ctx_rows.append(jnp.concatenate(heads, axis=1))
        ctx = jnp.concatenate(ctx_rows, axis=0)

        wo = wo_ref[pl.ds(my * DLOC, DLOC), :]
        part_ref[...] = jnp.dot(ctx, wo, preferred_element_type=jnp.float32)

        barrier_sem = pltpu.get_barrier_semaphore()
        for o in range(1, N_DEV):
            pl.semaphore_signal(
                barrier_sem, inc=1,
                device_id=(lax.rem(my + o, N_DEV),),
                device_id_type=pl.DeviceIdType.MESH,
            )
        pl.semaphore_wait(barrier_sem, N_DEV - 1)

        p1 = []
        for o in range(1, N_DEV):
            tgt = lax.rem(my + o, N_DEV)
            r = pltpu.make_async_remote_copy(
                src_ref=part_ref.at[pl.ds(tgt * CH, CH), :],
                dst_ref=p1_buf.at[o - 1],
                send_sem=p1_send.at[o - 1],
                recv_sem=p1_recv.at[o - 1],
                device_id=(tgt,),
                device_id_type=pl.DeviceIdType.MESH,
            )
            r.start()
            p1.append(r)

        red = part_ref[pl.ds(my * CH, CH), :]
        for o, r in enumerate(p1):
            r.wait_recv()
            red = red + p1_buf[o, :, :]
        red_ref[...] = red
        out_ref[pl.ds(my * CH, CH), :] = red

        p2 = []
        for o in range(1, N_DEV):
            tgt = lax.rem(my + o, N_DEV)
            r = pltpu.make_async_remote_copy(
                src_ref=red_ref,
                dst_ref=out_ref.at[pl.ds(my * CH, CH), :],
                send_sem=p2_send.at[o - 1],
                recv_sem=p2_recv.at[o - 1],
                device_id=(tgt,),
                device_id_type=pl.DeviceIdType.MESH,
            )
            r.start()
            p2.append(r)
        for r in p2:
            r.wait_recv()
        for r in p1:
            r.wait_send()
        for r in p2:
            r.wait_send()

    out2 = pl.pallas_call(
        body,
        out_shape=jax.ShapeDtypeStruct((ROWS, DM), jnp.float32),
        in_specs=[pl.BlockSpec(memory_space=pltpu.VMEM)] * 5,
        out_specs=pl.BlockSpec(memory_space=pltpu.VMEM),
        scratch_shapes=[
            pltpu.VMEM((ROWS, DM), jnp.float32),
            pltpu.VMEM((CH, DM), jnp.float32),
            pltpu.VMEM((N_DEV - 1, CH, DM), jnp.float32),
            pltpu.SemaphoreType.DMA((N_DEV - 1,)),
            pltpu.SemaphoreType.DMA((N_DEV - 1,)),
            pltpu.SemaphoreType.DMA((N_DEV - 1,)),
            pltpu.SemaphoreType.DMA((N_DEV - 1,)),
        ],
        compiler_params=pltpu.CompilerParams(collective_id=0),
    )(x2, Wq, k2, v2, Wo)
    return out2.reshape(B, SQ, DM)


# device time: 18597 ns/iter; 5.7015x vs baseline; 1.6991x over previous
import jax
import jax.numpy as jnp
from jax import lax
from jax.experimental import pallas as pl
from jax.experimental.pallas import tpu as pltpu

N_DEV = 8
B, SQ, DM = 2, 256, 512
HQ, DH = 32, 64
H_LOC = HQ // N_DEV
DLOC = H_LOC * DH
ROWS = B * SQ
CH = ROWS // N_DEV


def kernel(x, Wq, K_ext, V_ext, Wo):
    x2 = x.reshape(ROWS, DM)
    k2 = K_ext.reshape(ROWS, DLOC)
    v2 = V_ext.reshape(ROWS, DLOC)

    def body(x_ref, wq_ref, k_ref, v_ref, wo_ref, out_ref,
             send_buf, red_ref, p1_buf,
             p1_send, p1_recv, p2_send, p2_recv):
        my = lax.axis_index("i")

        barrier_sem = pltpu.get_barrier_semaphore()
        for o in range(1, N_DEV):
            pl.semaphore_signal(
                barrier_sem, inc=1,
                device_id=(lax.rem(my + o, N_DEV),),
                device_id_type=pl.DeviceIdType.MESH,
            )

        wq = wq_ref[:, pl.ds(my * DLOC, DLOC)]
        wo = wo_ref[pl.ds(my * DLOC, DLOC), :]

        def chunk(dd):
            r0 = dd * CH
            xc = x_ref[pl.ds(r0, CH), :]
            qc = jnp.dot(xc, wq, preferred_element_type=jnp.float32)
            kc = k_ref[pl.ds(r0, CH), :]
            vc = v_ref[pl.ds(r0, CH), :]
            heads = []
            for h in range(H_LOC):
                c0 = h * DH
                s = lax.dot_general(
                    qc[:, c0:c0 + DH], kc[:, c0:c0 + DH],
                    (((1,), (1,)), ((), ())),
                    preferred_element_type=jnp.float32,
                ) * 0.125
                m = jnp.max(s, axis=-1, keepdims=True)
                w = jnp.exp(s - m)
                w = w / jnp.sum(w, axis=-1, keepdims=True)
                heads.append(jnp.dot(w, vc[:, c0:c0 + DH],
                                     preferred_element_type=jnp.float32))
            ctx = jnp.concatenate(heads, axis=1)
            return jnp.dot(ctx, wo, preferred_element_type=jnp.float32)

        p1 = []
        for o in range(1, N_DEV):
            tgt = lax.rem(my + o, N_DEV)
            send_buf[o - 1, :, :] = chunk(tgt)
            if o == 1:
                pl.semaphore_wait(barrier_sem, N_DEV - 1)
            r = pltpu.make_async_remote_copy(
                src_ref=send_buf.at[o - 1],
                dst_ref=p1_buf.at[o - 1],
                send_sem=p1_send.at[o - 1],
                recv_sem=p1_recv.at[o - 1],
                device_id=(tgt,),
                device_id_type=pl.DeviceIdType.MESH,
            )
            r.start()
            p1.append(r)

        red = chunk(my)
        for o, r in enumerate(p1):
            r.wait_recv()
            red = red + p1_buf[o, :, :]
        red_ref[...] = red
        out_ref[pl.ds(my * CH, CH), :] = red

        p2 = []
        for o in range(1, N_DEV):
            tgt = lax.rem(my + o, N_DEV)
            r = pltpu.make_async_remote_copy(
                src_ref=red_ref,
                dst_ref=out_ref.at[pl.ds(my * CH, CH), :],
                send_sem=p2_send.at[o - 1],
                recv_sem=p2_recv.at[o - 1],
                device_id=(tgt,),
                device_id_type=pl.DeviceIdType.MESH,
            )
            r.start()
            p2.append(r)
        for r in p2:
            r.wait_recv()
        for r in p1:
            r.wait_send()
        for r in p2:
            r.wait_send()

    out2 = pl.pallas_call(
        body,
        out_shape=jax.ShapeDtypeStruct((ROWS, DM), jnp.float32),
        in_specs=[pl.BlockSpec(memory_space=pltpu.VMEM)] * 5,
        out_specs=pl.BlockSpec(memory_space=pltpu.VMEM),
        scratch_shapes=[
            pltpu.VMEM((N_DEV - 1, CH, DM), jnp.float32),
            pltpu.VMEM((CH, DM), jnp.float32),
            pltpu.VMEM((N_DEV - 1, CH, DM), jnp.float32),
            pltpu.SemaphoreType.DMA((N_DEV - 1,)),
            pltpu.SemaphoreType.DMA((N_DEV - 1,)),
            pltpu.SemaphoreType.DMA((N_DEV - 1,)),
            pltpu.SemaphoreType.DMA((N_DEV - 1,)),
        ],
        compiler_params=pltpu.CompilerParams(collective_id=0),
    )(x2, Wq, k2, v2, Wo)
    return out2.reshape(B, SQ, DM)
